# baseline (speedup 1.0000x reference)
import functools

import jax
import jax.numpy as jnp
from jax import lax
from jax.experimental import pallas as pl
from jax.experimental.pallas import tpu as pltpu
from jax.experimental.pallas import tpu_sc as plsc

_N = 10000
_E = 320000
_D = 128
_H = 128

_CH = 128
_NW = 32
_NCHUNK = 2560
_EPAD = _NCHUNK * _CH
_CPT = _NCHUNK // _NW
_CA = 80
_CB = 160 - _CA
_IDXPAD = _NCHUNK + _CA
_NPAD = 10112
_RPS = _NPAD // 16

_BE = 4000
_GE = _E // _BE

_SQRT2 = 1.4142135623730951


def _gelu(x):
    return 0.5 * x * (1.0 + lax.erf(x / _SQRT2))


def _pack2(hi_f32, lo_f32):
    hi = lax.bitcast_convert_type(hi_f32, jnp.uint32)
    lo = lax.bitcast_convert_type(lo_f32, jnp.uint32)
    hi_r = (hi + jnp.uint32(0x7FFF) + ((hi >> 16) & jnp.uint32(1))) \
        & jnp.uint32(0xFFFF0000)
    lo_r = (lo + jnp.uint32(0x7FFF) + ((lo >> 16) & jnp.uint32(1))) >> 16
    return lax.bitcast_convert_type(hi_r | lo_r, jnp.float32)


def _unpack_hi(u):
    return lax.bitcast_convert_type(u & jnp.uint32(0xFFFF0000), jnp.float32)


def _unpack_lo(u):
    return lax.bitcast_convert_type(u << 16, jnp.float32)


def _node_pre_body(h_ref, wsT_ref, bs_ref, wdT_ref, bd_ref, wtdT_ref, btd_ref,
                   tsrc_ref, td_ref, tdst_ref):
    h = h_ref[...]
    hs = jnp.dot(h, wsT_ref[...], preferred_element_type=jnp.float32) + bs_ref[...]
    tsrc_ref[...] = _pack2(hs, h)
    hd = jnp.dot(h, wdT_ref[...], preferred_element_type=jnp.float32) + bd_ref[...]
    td_ref[...] = hd
    tdst_ref[...] = (
        jnp.dot(h, wtdT_ref[...], preferred_element_type=jnp.float32) + btd_ref[...]
    )


def _node_pre(h, wsT, bs, wdT, bd, wtdT, btd):
    return pl.pallas_call(
        _node_pre_body,
        out_shape=[
            jax.ShapeDtypeStruct((_N, _H), jnp.float32),
            jax.ShapeDtypeStruct((_N, _H), jnp.float32),
            jax.ShapeDtypeStruct((_N, _H), jnp.float32),
        ],
    )(h, wsT, bs, wdT, bd, wtdT, btd)


def _edge_pre_body(ea_ref, w1T_ref, b1_ref, w2T_ref, b2_ref, ee_ref):
    ea = ea_ref[...]
    e1 = _gelu(jnp.dot(ea, w1T_ref[...], preferred_element_type=jnp.float32)
               + b1_ref[...])
    ee = jnp.dot(e1, w2T_ref[...], preferred_element_type=jnp.float32) + b2_ref[...]
    ee_ref[...] = ee.astype(jnp.bfloat16)


def _edge_pre(ea, w1T, b1, w2T, b2):
    wspec = pl.BlockSpec((_H, _H), lambda i: (0, 0))
    bspec = pl.BlockSpec((1, _H), lambda i: (0, 0))
    return pl.pallas_call(
        _edge_pre_body,
        grid=(_GE,),
        in_specs=[
            pl.BlockSpec((_BE, _D), lambda i: (i, 0)),
            wspec, bspec, wspec, bspec,
        ],
        out_specs=pl.BlockSpec((_BE, _H), lambda i: (i, 0)),
        out_shape=jax.ShapeDtypeStruct((_EPAD, _H), jnp.bfloat16),
    )(ea, w1T, b1, w2T, b2)


def _edge_body(ee_ref, gsrc_ref, gd_ref,
               p1T_ref, pb1_ref, p2T_ref, pb2_ref, msgs_ref):
    ee = ee_ref[...].astype(jnp.float32)
    us = lax.bitcast_convert_type(gsrc_ref[...], jnp.uint32)
    hs_g = _unpack_hi(us)
    h_g = _unpack_lo(us)
    hd_g = gd_ref[...]
    z = _gelu(ee + hs_g + hd_g)
    z = _gelu(jnp.dot(z, p1T_ref[...], preferred_element_type=jnp.float32)
              + pb1_ref[...])
    he = jnp.dot(z, p2T_ref[...], preferred_element_type=jnp.float32) + pb2_ref[...]
    msgs_ref[...] = h_g * he


def _edge_mlp(ee, gsrc, gd, p1T, pb1, p2T, pb2):
    wspec = pl.BlockSpec((_H, _H), lambda i: (0, 0))
    bspec = pl.BlockSpec((1, _H), lambda i: (0, 0))
    return pl.pallas_call(
        _edge_body,
        grid=(_GE,),
        in_specs=[
            pl.BlockSpec((_BE, _H), lambda i: (i, 0)),
            pl.BlockSpec((_BE, _H), lambda i: (i, 0)),
            pl.BlockSpec((_BE, _H), lambda i: (i, 0)),
            wspec, bspec, wspec, bspec,
        ],
        out_specs=pl.BlockSpec((_BE, _H), lambda i: (i, 0)),
        out_shape=jax.ShapeDtypeStruct((_EPAD, _H), jnp.float32),
    )(ee, gsrc, gd, p1T, pb1, p2T, pb2)


def _node_fin_body(tdst_ref, mp_ref, tmT_ref, tmb_ref, t1T_ref, t1b_ref, out_ref):
    m = mp_ref[0:_N, :] + mp_ref[_NPAD:_NPAD + _N, :]
    t = (tdst_ref[...]
         + jnp.dot(m, tmT_ref[...], preferred_element_type=jnp.float32)
         + tmb_ref[...])
    u = _gelu(t)
    out_ref[...] = _gelu(
        jnp.dot(u, t1T_ref[...], preferred_element_type=jnp.float32) + t1b_ref[...]
    )


def _node_fin(tdst, mp, tmT, tmb, t1T, t1b):
    return pl.pallas_call(
        _node_fin_body,
        out_shape=jax.ShapeDtypeStruct((_N, _H), jnp.float32),
    )(tdst, mp, tmT, tmb, t1T, t1b)


@functools.cache
def _build_sc_kernels():
    mesh = plsc.VectorSubcoreMesh(core_axis_name="c", subcore_axis_name="s")

    _NB = 2

    @functools.partial(
        pl.kernel,
        out_type=[
            jax.ShapeDtypeStruct((_EPAD, _H), jnp.float32),
            jax.ShapeDtypeStruct((_EPAD, _H), jnp.float32),
        ],
        mesh=mesh,
        scratch_types=(
            [pltpu.VMEM((_CA, _CH), jnp.int32)] * 2
            + [pltpu.VMEM((_CH, _H), jnp.float32)] * _NB
            + [pltpu.VMEM((_CH, _H), jnp.float32)] * _NB
            + [pltpu.SemaphoreType.DMA] * (4 * _NB)
        ),
    )
    def _sc_gather(tsrc_hbm, td_hbm, src_hbm, dst_hbm, gsrc_hbm, gd_hbm,
                   *bufs):
        idxs_v, idxd_v = bufs[0], bufs[1]
        rs = bufs[2:2 + _NB]
        rd = bufs[2 + _NB:2 + 2 * _NB]
        sems = bufs[2 + 2 * _NB:]
        sgs = sems[0:_NB]
        sgd = sems[_NB:2 * _NB]
        sws = sems[2 * _NB:3 * _NB]
        swd = sems[3 * _NB:4 * _NB]
        c = lax.axis_index("c")
        s = lax.axis_index("s")
        cnt = jnp.where(c == 0, _CA, _CB)
        base = jnp.where(c == 0, s * _CA, 16 * _CA + s * _CB)
        pltpu.sync_copy(src_hbm.at[pl.ds(base, _CA)], idxs_v)
        pltpu.sync_copy(dst_hbm.at[pl.ds(base, _CA)], idxd_v)

        for p in range(_NB):
            pltpu.async_copy(tsrc_hbm.at[idxs_v.at[p]], rs[p], sgs[p])
            pltpu.async_copy(td_hbm.at[idxd_v.at[p]], rd[p], sgd[p])

        def outer(k, carry):
            for p in range(_NB):
                i = _NB * k + p
                row0 = (base + i) * _CH
                pltpu.make_async_copy(
                    tsrc_hbm.at[idxs_v.at[i]], rs[p], sgs[p]).wait()
                pltpu.make_async_copy(
                    td_hbm.at[idxd_v.at[i]], rd[p], sgd[p]).wait()
                pltpu.async_copy(rs[p], gsrc_hbm.at[pl.ds(row0, _CH)], sws[p])
                pltpu.async_copy(rd[p], gd_hbm.at[pl.ds(row0, _CH)], swd[p])

                @pl.when(i + _NB < cnt)
                def _():
                    pltpu.make_async_copy(
                        rs[p], gsrc_hbm.at[pl.ds(row0, _CH)], sws[p]).wait()
                    pltpu.make_async_copy(
                        rd[p], gd_hbm.at[pl.ds(row0, _CH)], swd[p]).wait()
                    pltpu.async_copy(
                        tsrc_hbm.at[idxs_v.at[i + _NB]], rs[p], sgs[p])
                    pltpu.async_copy(
                        td_hbm.at[idxd_v.at[i + _NB]], rd[p], sgd[p])
            return carry

        lax.fori_loop(0, cnt // _NB, outer, 0)
        for p in range(_NB):
            row0 = (base + cnt - _NB + p) * _CH
            pltpu.make_async_copy(
                rs[p], gsrc_hbm.at[pl.ds(row0, _CH)], sws[p]).wait()
            pltpu.make_async_copy(
                rd[p], gd_hbm.at[pl.ds(row0, _CH)], swd[p]).wait()

    @functools.partial(
        pl.kernel,
        out_type=jax.ShapeDtypeStruct((2 * _NPAD, _H), jnp.float32),
        mesh=mesh,
        scratch_types=[
            pltpu.VMEM((_CPT, _CH), jnp.int32),
            pltpu.VMEM((_CH, _H), jnp.float32),
            pltpu.VMEM_SHARED((_NPAD, _H), jnp.float32),
        ],
    )
    def _sc_scatter(msgs_hbm, dst_hbm, zeros_hbm, out_hbm, idx_v, rows_v, acc_sh):
        c = lax.axis_index("c")
        s = lax.axis_index("s")
        pltpu.sync_copy(zeros_hbm, acc_sh.at[pl.ds(s * _RPS, _RPS)])
        plsc.subcore_barrier()
        base = c * (_NCHUNK // 2) + s * _CPT
        pltpu.sync_copy(dst_hbm.at[pl.ds(base, _CPT)], idx_v)

        def body(i, carry):
            j = base + i
            pltpu.sync_copy(msgs_hbm.at[pl.ds(j * _CH, _CH)], rows_v)
            pltpu.sync_copy(rows_v, acc_sh.at[idx_v.at[i]], add=True)
            return carry

        lax.fori_loop(0, _CPT, body, 0)
        plsc.subcore_barrier()
        pltpu.sync_copy(acc_sh.at[pl.ds(s * _RPS, _RPS)],
                        out_hbm.at[pl.ds(c * _NPAD + s * _RPS, _RPS)])

    return _sc_gather, _sc_scatter


def kernel(node_feats, edge_index, edge_attr, params):
    p = params
    src = edge_index[0]
    dst = edge_index[1]
    pad = _IDXPAD * _CH - _E
    srcp = jnp.concatenate(
        [src, jnp.zeros((pad,), jnp.int32)]).reshape(_IDXPAD, _CH)
    dstp = jnp.concatenate(
        [dst, jnp.full((pad,), _N, jnp.int32)]).reshape(_IDXPAD, _CH)

    def b(v):
        return v.reshape(1, _H)

    sc_gather, sc_scatter = _build_sc_kernels()
    tsrc, td, tdst = _node_pre(node_feats, p["ans_W"].T, b(p["ans_b"]),
                               p["and_W"].T, b(p["and_b"]),
                               p["td_W"].T, b(p["td_b"]))
    gsrc, gd = sc_gather(tsrc, td, srcp, dstp)
    ee = _edge_pre(edge_attr, p["ae_W1"].T, b(p["ae_b1"]),
                   p["ae_W2"].T, b(p["ae_b2"]))
    msgs = _edge_mlp(ee, gsrc, gd,
                     p["phi_W1"].T, b(p["phi_b1"]), p["phi_W2"].T, b(p["phi_b2"]))
    zeros = jnp.zeros((_RPS, _H), jnp.float32)
    mp = sc_scatter(msgs, dstp, zeros)
    out = _node_fin(tdst, mp, p["tm_W"].T, b(p["tm_b"]),
                    p["t_W1"].T, b(p["t_b1"]))
    return out


# speedup vs baseline: 1.0308x; 1.0308x over previous
import functools

import jax
import jax.numpy as jnp
from jax import lax
from jax.experimental import pallas as pl
from jax.experimental.pallas import tpu as pltpu
from jax.experimental.pallas import tpu_sc as plsc

_N = 10000
_E = 320000
_D = 128
_H = 128

_CH = 128
_NW = 32
_NCHUNK = 2560
_EPAD = _NCHUNK * _CH
_CPT = _NCHUNK // _NW
_CA = 80
_CB = 160 - _CA
_IDXPAD = _NCHUNK + _CA
_NPAD = 10112
_RPS = _NPAD // 16

_BE = 4000
_GE = _E // _BE

_SQRT2 = 1.4142135623730951


def _gelu(x):
    return 0.5 * x * (1.0 + lax.erf(x / _SQRT2))


def _pack2(hi_f32, lo_f32):
    hi = lax.bitcast_convert_type(hi_f32, jnp.uint32)
    lo = lax.bitcast_convert_type(lo_f32, jnp.uint32)
    hi_r = (hi + jnp.uint32(0x7FFF) + ((hi >> 16) & jnp.uint32(1))) \
        & jnp.uint32(0xFFFF0000)
    lo_r = (lo + jnp.uint32(0x7FFF) + ((lo >> 16) & jnp.uint32(1))) >> 16
    return lax.bitcast_convert_type(hi_r | lo_r, jnp.float32)


def _unpack_hi(u):
    return lax.bitcast_convert_type(u & jnp.uint32(0xFFFF0000), jnp.float32)


def _unpack_lo(u):
    return lax.bitcast_convert_type(u << 16, jnp.float32)


def _node_pre_body(h_ref, wsT_ref, bs_ref, wdT_ref, bd_ref, wtdT_ref, btd_ref,
                   tsrc_ref, td_ref, tdst_ref):
    h = h_ref[...]
    hs = jnp.dot(h, wsT_ref[...], preferred_element_type=jnp.float32) + bs_ref[...]
    tsrc_ref[...] = _pack2(hs, h)
    hd = jnp.dot(h, wdT_ref[...], preferred_element_type=jnp.float32) + bd_ref[...]
    td_ref[...] = hd
    tdst_ref[...] = (
        jnp.dot(h, wtdT_ref[...], preferred_element_type=jnp.float32) + btd_ref[...]
    )


def _node_pre(h, wsT, bs, wdT, bd, wtdT, btd):
    return pl.pallas_call(
        _node_pre_body,
        out_shape=[
            jax.ShapeDtypeStruct((_N, _H), jnp.float32),
            jax.ShapeDtypeStruct((_N, _H), jnp.float32),
            jax.ShapeDtypeStruct((_N, _H), jnp.float32),
        ],
    )(h, wsT, bs, wdT, bd, wtdT, btd)


def _edge_body(ea_ref, gsrc_ref, gd_ref, w1T_ref, b1_ref, w2T_ref, b2_ref,
               p1T_ref, pb1_ref, p2T_ref, pb2_ref, msgs_ref):
    ea = ea_ref[...]
    e1 = _gelu(jnp.dot(ea, w1T_ref[...], preferred_element_type=jnp.float32)
               + b1_ref[...])
    ee = jnp.dot(e1, w2T_ref[...], preferred_element_type=jnp.float32) + b2_ref[...]
    us = lax.bitcast_convert_type(gsrc_ref[...], jnp.uint32)
    hs_g = _unpack_hi(us)
    h_g = _unpack_lo(us)
    z = _gelu(ee + hs_g + gd_ref[...])
    z = _gelu(jnp.dot(z, p1T_ref[...], preferred_element_type=jnp.float32)
              + pb1_ref[...])
    he = jnp.dot(z, p2T_ref[...], preferred_element_type=jnp.float32) + pb2_ref[...]
    msgs_ref[...] = h_g * he


def _edge_mlp(ea, gsrc, gd, w1T, b1, w2T, b2, p1T, pb1, p2T, pb2):
    wspec = pl.BlockSpec((_H, _H), lambda i: (0, 0))
    bspec = pl.BlockSpec((1, _H), lambda i: (0, 0))
    return pl.pallas_call(
        _edge_body,
        grid=(_GE,),
        in_specs=[
            pl.BlockSpec((_BE, _D), lambda i: (i, 0)),
            pl.BlockSpec((_BE, _H), lambda i: (i, 0)),
            pl.BlockSpec((_BE, _H), lambda i: (i, 0)),
            wspec, bspec, wspec, bspec, wspec, bspec, wspec, bspec,
        ],
        out_specs=pl.BlockSpec((_BE, _H), lambda i: (i, 0)),
        out_shape=jax.ShapeDtypeStruct((_EPAD, _H), jnp.float32),
    )(ea, gsrc, gd, w1T, b1, w2T, b2, p1T, pb1, p2T, pb2)


def _node_fin_body(tdst_ref, mp_ref, tmT_ref, tmb_ref, t1T_ref, t1b_ref, out_ref):
    m = mp_ref[0:_N, :] + mp_ref[_NPAD:_NPAD + _N, :]
    t = (tdst_ref[...]
         + jnp.dot(m, tmT_ref[...], preferred_element_type=jnp.float32)
         + tmb_ref[...])
    u = _gelu(t)
    out_ref[...] = _gelu(
        jnp.dot(u, t1T_ref[...], preferred_element_type=jnp.float32) + t1b_ref[...]
    )


def _node_fin(tdst, mp, tmT, tmb, t1T, t1b):
    return pl.pallas_call(
        _node_fin_body,
        out_shape=jax.ShapeDtypeStruct((_N, _H), jnp.float32),
    )(tdst, mp, tmT, tmb, t1T, t1b)


@functools.cache
def _build_sc_kernels():
    mesh = plsc.VectorSubcoreMesh(core_axis_name="c", subcore_axis_name="s")

    _NB = 2

    @functools.partial(
        pl.kernel,
        out_type=[
            jax.ShapeDtypeStruct((_EPAD, _H), jnp.float32),
            jax.ShapeDtypeStruct((_EPAD, _H), jnp.float32),
        ],
        mesh=mesh,
        scratch_types=(
            [pltpu.VMEM((_CPT, _CH), jnp.int32)] * 2
            + [pltpu.VMEM((_CH, _H), jnp.float32)] * _NB
            + [pltpu.VMEM((_CH, _H), jnp.float32)] * _NB
            + [pltpu.SemaphoreType.DMA] * (4 * _NB)
        ),
    )
    def _sc_gather(tsrc_hbm, td_hbm, src_hbm, dst_hbm, gsrc_hbm, gd_hbm,
                   *bufs):
        idxs_v, idxd_v = bufs[0], bufs[1]
        rs = bufs[2:2 + _NB]
        rd = bufs[2 + _NB:2 + 2 * _NB]
        sems = bufs[2 + 2 * _NB:]
        sgs = sems[0:_NB]
        sgd = sems[_NB:2 * _NB]
        sws = sems[2 * _NB:3 * _NB]
        swd = sems[3 * _NB:4 * _NB]
        w = lax.axis_index("s") * 2 + lax.axis_index("c")
        base = w * _CPT
        pltpu.sync_copy(src_hbm.at[pl.ds(base, _CPT)], idxs_v)
        pltpu.sync_copy(dst_hbm.at[pl.ds(base, _CPT)], idxd_v)

        for p in range(_NB):
            pltpu.async_copy(tsrc_hbm.at[idxs_v.at[p]], rs[p], sgs[p])
            pltpu.async_copy(td_hbm.at[idxd_v.at[p]], rd[p], sgd[p])

        def outer(k, carry):
            for p in range(_NB):
                i = _NB * k + p
                row0 = (base + i) * _CH
                pltpu.make_async_copy(
                    tsrc_hbm.at[idxs_v.at[i]], rs[p], sgs[p]).wait()
                pltpu.make_async_copy(
                    td_hbm.at[idxd_v.at[i]], rd[p], sgd[p]).wait()
                pltpu.async_copy(rs[p], gsrc_hbm.at[pl.ds(row0, _CH)], sws[p])
                pltpu.async_copy(rd[p], gd_hbm.at[pl.ds(row0, _CH)], swd[p])

                @pl.when(i + _NB < _CPT)
                def _():
                    pltpu.make_async_copy(
                        rs[p], gsrc_hbm.at[pl.ds(row0, _CH)], sws[p]).wait()
                    pltpu.make_async_copy(
                        rd[p], gd_hbm.at[pl.ds(row0, _CH)], swd[p]).wait()
                    pltpu.async_copy(
                        tsrc_hbm.at[idxs_v.at[i + _NB]], rs[p], sgs[p])
                    pltpu.async_copy(
                        td_hbm.at[idxd_v.at[i + _NB]], rd[p], sgd[p])
            return carry

        lax.fori_loop(0, _CPT // _NB, outer, 0)
        for p in range(_NB):
            row0 = (base + _CPT - _NB + p) * _CH
            pltpu.make_async_copy(
                rs[p], gsrc_hbm.at[pl.ds(row0, _CH)], sws[p]).wait()
            pltpu.make_async_copy(
                rd[p], gd_hbm.at[pl.ds(row0, _CH)], swd[p]).wait()

    @functools.partial(
        pl.kernel,
        out_type=jax.ShapeDtypeStruct((2 * _NPAD, _H), jnp.float32),
        mesh=mesh,
        scratch_types=[
            pltpu.VMEM((_CPT, _CH), jnp.int32),
            pltpu.VMEM((_CH, _H), jnp.float32),
            pltpu.VMEM_SHARED((_NPAD, _H), jnp.float32),
        ],
    )
    def _sc_scatter(msgs_hbm, dst_hbm, zeros_hbm, out_hbm, idx_v, rows_v, acc_sh):
        c = lax.axis_index("c")
        s = lax.axis_index("s")
        pltpu.sync_copy(zeros_hbm, acc_sh.at[pl.ds(s * _RPS, _RPS)])
        plsc.subcore_barrier()
        base = c * (_NCHUNK // 2) + s * _CPT
        pltpu.sync_copy(dst_hbm.at[pl.ds(base, _CPT)], idx_v)

        def body(i, carry):
            j = base + i
            pltpu.sync_copy(msgs_hbm.at[pl.ds(j * _CH, _CH)], rows_v)
            pltpu.sync_copy(rows_v, acc_sh.at[idx_v.at[i]], add=True)
            return carry

        lax.fori_loop(0, _CPT, body, 0)
        plsc.subcore_barrier()
        pltpu.sync_copy(acc_sh.at[pl.ds(s * _RPS, _RPS)],
                        out_hbm.at[pl.ds(c * _NPAD + s * _RPS, _RPS)])

    return _sc_gather, _sc_scatter


def kernel(node_feats, edge_index, edge_attr, params):
    p = params
    src = edge_index[0]
    dst = edge_index[1]
    pad = _IDXPAD * _CH - _E
    srcp = jnp.concatenate(
        [src, jnp.zeros((pad,), jnp.int32)]).reshape(_IDXPAD, _CH)
    dstp = jnp.concatenate(
        [dst, jnp.full((pad,), _N, jnp.int32)]).reshape(_IDXPAD, _CH)

    def b(v):
        return v.reshape(1, _H)

    sc_gather, sc_scatter = _build_sc_kernels()
    tsrc, td, tdst = _node_pre(node_feats, p["ans_W"].T, b(p["ans_b"]),
                               p["and_W"].T, b(p["and_b"]),
                               p["td_W"].T, b(p["td_b"]))
    gsrc, gd = sc_gather(tsrc, td, srcp, dstp)
    msgs = _edge_mlp(edge_attr, gsrc, gd,
                     p["ae_W1"].T, b(p["ae_b1"]), p["ae_W2"].T, b(p["ae_b2"]),
                     p["phi_W1"].T, b(p["phi_b1"]), p["phi_W2"].T, b(p["phi_b2"]))
    zeros = jnp.zeros((_RPS, _H), jnp.float32)
    mp = sc_scatter(msgs, dstp, zeros)
    out = _node_fin(tdst, mp, p["tm_W"].T, b(p["tm_b"]),
                    p["t_W1"].T, b(p["t_b1"]))
    return out


# speedup vs baseline: 1.0595x; 1.0278x over previous
import functools

import jax
import jax.numpy as jnp
from jax import lax
from jax.experimental import pallas as pl
from jax.experimental.pallas import tpu as pltpu
from jax.experimental.pallas import tpu_sc as plsc

_N = 10000
_E = 320000
_D = 128
_H = 128

_CH = 128
_NW = 32
_NCHUNK = 2560
_EPAD = _NCHUNK * _CH
_CPT = _NCHUNK // _NW
_IDXPAD = _NCHUNK + _CPT
_NPAD = 10112
_RPS = _NPAD // 16

_BE = 8000
_GE = _E // _BE

_SQRT2 = 1.4142135623730951


def _gelu(x):
    return 0.5 * x * (1.0 + lax.erf(x / _SQRT2))


def _pack2(hi_f32, lo_f32):
    hi = lax.bitcast_convert_type(hi_f32, jnp.uint32)
    lo = lax.bitcast_convert_type(lo_f32, jnp.uint32)
    hi_r = (hi + jnp.uint32(0x7FFF) + ((hi >> 16) & jnp.uint32(1))) \
        & jnp.uint32(0xFFFF0000)
    lo_r = (lo + jnp.uint32(0x7FFF) + ((lo >> 16) & jnp.uint32(1))) >> 16
    return lax.bitcast_convert_type(hi_r | lo_r, jnp.float32)


def _unpack_hi(u):
    return lax.bitcast_convert_type(u & jnp.uint32(0xFFFF0000), jnp.float32)


def _unpack_lo(u):
    return lax.bitcast_convert_type(u << 16, jnp.float32)


def _node_pre_body(h_ref, wsT_ref, bs_ref, wdT_ref, bd_ref, wtdT_ref, btd_ref,
                   tsrc_ref, td_ref, tdst_ref):
    h = h_ref[...]
    hs = jnp.dot(h, wsT_ref[...], preferred_element_type=jnp.float32) + bs_ref[...]
    tsrc_ref[...] = _pack2(hs, h)
    hd = jnp.dot(h, wdT_ref[...], preferred_element_type=jnp.float32) + bd_ref[...]
    td_ref[...] = hd
    tdst_ref[...] = (
        jnp.dot(h, wtdT_ref[...], preferred_element_type=jnp.float32) + btd_ref[...]
    )


def _node_pre(h, wsT, bs, wdT, bd, wtdT, btd):
    return pl.pallas_call(
        _node_pre_body,
        out_shape=[
            jax.ShapeDtypeStruct((_N, _H), jnp.float32),
            jax.ShapeDtypeStruct((_N, _H), jnp.float32),
            jax.ShapeDtypeStruct((_N, _H), jnp.float32),
        ],
    )(h, wsT, bs, wdT, bd, wtdT, btd)


def _edge_body(ea_ref, gsrc_ref, gd_ref, w1T_ref, b1_ref, w2T_ref, b2_ref,
               p1T_ref, pb1_ref, p2T_ref, pb2_ref, msgs_ref):
    ea = ea_ref[...]
    e1 = _gelu(jnp.dot(ea, w1T_ref[...], preferred_element_type=jnp.float32)
               + b1_ref[...])
    ee = jnp.dot(e1, w2T_ref[...], preferred_element_type=jnp.float32) + b2_ref[...]
    us = lax.bitcast_convert_type(gsrc_ref[...], jnp.uint32)
    hs_g = _unpack_hi(us)
    h_g = _unpack_lo(us)
    z = _gelu(ee + hs_g + gd_ref[...])
    z = _gelu(jnp.dot(z, p1T_ref[...], preferred_element_type=jnp.float32)
              + pb1_ref[...])
    he = jnp.dot(z, p2T_ref[...], preferred_element_type=jnp.float32) + pb2_ref[...]
    msgs_ref[...] = h_g * he


def _edge_mlp(ea, gsrc, gd, w1T, b1, w2T, b2, p1T, pb1, p2T, pb2):
    wspec = pl.BlockSpec((_H, _H), lambda i: (0, 0))
    bspec = pl.BlockSpec((1, _H), lambda i: (0, 0))
    return pl.pallas_call(
        _edge_body,
        grid=(_GE,),
        in_specs=[
            pl.BlockSpec((_BE, _D), lambda i: (i, 0)),
            pl.BlockSpec((_BE, _H), lambda i: (i, 0)),
            pl.BlockSpec((_BE, _H), lambda i: (i, 0)),
            wspec, bspec, wspec, bspec, wspec, bspec, wspec, bspec,
        ],
        out_specs=pl.BlockSpec((_BE, _H), lambda i: (i, 0)),
        out_shape=jax.ShapeDtypeStruct((_EPAD, _H), jnp.float32),
    )(ea, gsrc, gd, w1T, b1, w2T, b2, p1T, pb1, p2T, pb2)


def _node_fin_body(tdst_ref, mp_ref, tmT_ref, tmb_ref, t1T_ref, t1b_ref, out_ref):
    m = mp_ref[0:_N, :] + mp_ref[_NPAD:_NPAD + _N, :]
    t = (tdst_ref[...]
         + jnp.dot(m, tmT_ref[...], preferred_element_type=jnp.float32)
         + tmb_ref[...])
    u = _gelu(t)
    out_ref[...] = _gelu(
        jnp.dot(u, t1T_ref[...], preferred_element_type=jnp.float32) + t1b_ref[...]
    )


def _node_fin(tdst, mp, tmT, tmb, t1T, t1b):
    return pl.pallas_call(
        _node_fin_body,
        out_shape=jax.ShapeDtypeStruct((_N, _H), jnp.float32),
    )(tdst, mp, tmT, tmb, t1T, t1b)


@functools.cache
def _build_sc_kernels():
    mesh = plsc.VectorSubcoreMesh(core_axis_name="c", subcore_axis_name="s")

    _NB = 2

    @functools.partial(
        pl.kernel,
        out_type=[
            jax.ShapeDtypeStruct((_EPAD, _H), jnp.float32),
            jax.ShapeDtypeStruct((_EPAD, _H), jnp.float32),
        ],
        mesh=mesh,
        scratch_types=(
            [pltpu.VMEM((_CPT, _CH), jnp.int32)] * 2
            + [pltpu.VMEM((_CH, _H), jnp.float32)] * _NB
            + [pltpu.VMEM((_CH, _H), jnp.float32)] * _NB
            + [pltpu.SemaphoreType.DMA] * (4 * _NB)
        ),
    )
    def _sc_gather(tsrc_hbm, td_hbm, src_hbm, dst_hbm, gsrc_hbm, gd_hbm,
                   *bufs):
        idxs_v, idxd_v = bufs[0], bufs[1]
        rs = bufs[2:2 + _NB]
        rd = bufs[2 + _NB:2 + 2 * _NB]
        sems = bufs[2 + 2 * _NB:]
        sgs = sems[0:_NB]
        sgd = sems[_NB:2 * _NB]
        sws = sems[2 * _NB:3 * _NB]
        swd = sems[3 * _NB:4 * _NB]
        w = lax.axis_index("s") * 2 + lax.axis_index("c")
        base = w * _CPT
        pltpu.sync_copy(src_hbm.at[pl.ds(base, _CPT)], idxs_v)
        pltpu.sync_copy(dst_hbm.at[pl.ds(base, _CPT)], idxd_v)

        for p in range(_NB):
            pltpu.async_copy(tsrc_hbm.at[idxs_v.at[p]], rs[p], sgs[p])
            pltpu.async_copy(td_hbm.at[idxd_v.at[p]], rd[p], sgd[p])

        def outer(k, carry):
            for p in range(_NB):
                i = _NB * k + p
                row0 = (base + i) * _CH
                pltpu.make_async_copy(
                    tsrc_hbm.at[idxs_v.at[i]], rs[p], sgs[p]).wait()
                pltpu.make_async_copy(
                    td_hbm.at[idxd_v.at[i]], rd[p], sgd[p]).wait()
                pltpu.async_copy(rs[p], gsrc_hbm.at[pl.ds(row0, _CH)], sws[p])
                pltpu.async_copy(rd[p], gd_hbm.at[pl.ds(row0, _CH)], swd[p])

                @pl.when(i + _NB < _CPT)
                def _():
                    pltpu.make_async_copy(
                        rs[p], gsrc_hbm.at[pl.ds(row0, _CH)], sws[p]).wait()
                    pltpu.make_async_copy(
                        rd[p], gd_hbm.at[pl.ds(row0, _CH)], swd[p]).wait()
                    pltpu.async_copy(
                        tsrc_hbm.at[idxs_v.at[i + _NB]], rs[p], sgs[p])
                    pltpu.async_copy(
                        td_hbm.at[idxd_v.at[i + _NB]], rd[p], sgd[p])
            return carry

        lax.fori_loop(0, _CPT // _NB, outer, 0)
        for p in range(_NB):
            row0 = (base + _CPT - _NB + p) * _CH
            pltpu.make_async_copy(
                rs[p], gsrc_hbm.at[pl.ds(row0, _CH)], sws[p]).wait()
            pltpu.make_async_copy(
                rd[p], gd_hbm.at[pl.ds(row0, _CH)], swd[p]).wait()

    @functools.partial(
        pl.kernel,
        out_type=jax.ShapeDtypeStruct((2 * _NPAD, _H), jnp.float32),
        mesh=mesh,
        scratch_types=[
            pltpu.VMEM((_CPT, _CH), jnp.int32),
            pltpu.VMEM((_CH, _H), jnp.float32),
            pltpu.VMEM_SHARED((_NPAD, _H), jnp.float32),
        ],
    )
    def _sc_scatter(msgs_hbm, dst_hbm, zeros_hbm, out_hbm, idx_v, rows_v, acc_sh):
        c = lax.axis_index("c")
        s = lax.axis_index("s")
        pltpu.sync_copy(zeros_hbm, acc_sh.at[pl.ds(s * _RPS, _RPS)])
        plsc.subcore_barrier()
        base = c * (_NCHUNK // 2) + s * _CPT
        pltpu.sync_copy(dst_hbm.at[pl.ds(base, _CPT)], idx_v)

        def body(i, carry):
            j = base + i
            pltpu.sync_copy(msgs_hbm.at[pl.ds(j * _CH, _CH)], rows_v)
            pltpu.sync_copy(rows_v, acc_sh.at[idx_v.at[i]], add=True)
            return carry

        lax.fori_loop(0, _CPT, body, 0)
        plsc.subcore_barrier()
        pltpu.sync_copy(acc_sh.at[pl.ds(s * _RPS, _RPS)],
                        out_hbm.at[pl.ds(c * _NPAD + s * _RPS, _RPS)])

    return _sc_gather, _sc_scatter


def kernel(node_feats, edge_index, edge_attr, params):
    p = params
    src = edge_index[0]
    dst = edge_index[1]
    pad = _IDXPAD * _CH - _E
    srcp = jnp.concatenate(
        [src, jnp.zeros((pad,), jnp.int32)]).reshape(_IDXPAD, _CH)
    dstp = jnp.concatenate(
        [dst, jnp.full((pad,), _N, jnp.int32)]).reshape(_IDXPAD, _CH)

    def b(v):
        return v.reshape(1, _H)

    sc_gather, sc_scatter = _build_sc_kernels()
    tsrc, td, tdst = _node_pre(node_feats, p["ans_W"].T, b(p["ans_b"]),
                               p["and_W"].T, b(p["and_b"]),
                               p["td_W"].T, b(p["td_b"]))
    gsrc, gd = sc_gather(tsrc, td, srcp, dstp)
    msgs = _edge_mlp(edge_attr, gsrc, gd,
                     p["ae_W1"].T, b(p["ae_b1"]), p["ae_W2"].T, b(p["ae_b2"]),
                     p["phi_W1"].T, b(p["phi_b1"]), p["phi_W2"].T, b(p["phi_b2"]))
    zeros = jnp.zeros((_RPS, _H), jnp.float32)
    mp = sc_scatter(msgs, dstp, zeros)
    out = _node_fin(tdst, mp, p["tm_W"].T, b(p["tm_b"]),
                    p["t_W1"].T, b(p["t_b1"]))
    return out
